# baseline (speedup 1.0000x reference)
import functools

import jax
import jax.numpy as jnp
from jax import lax
from jax.experimental import pallas as pl
from jax.experimental.pallas import tpu as pltpu

_EPS = 1e-5
_LANE = 128
_VMEM_LIMIT = 60 * 1024 * 1024
_NODE_TILE = 1024
_EDGE_TILE = 1024
_GATHER_UNROLL = 8


def _round_up(x, m):
    return ((x + m - 1) // m) * m


def _pad2d(x, rows=None, cols=None):
    r = 0 if rows is None else rows - x.shape[0]
    c = 0 if cols is None else cols - x.shape[1]
    if r == 0 and c == 0:
        return x
    return jnp.pad(x, ((0, r), (0, c)))


def _layernorm_relu(y, g, b, d_true):
    red_axes = tuple(range(1, y.ndim))
    Dp = 1
    for a in red_axes:
        Dp *= y.shape[a]
    inv_d = 1.0 / float(d_true)
    mean = jnp.sum(y, axis=red_axes, keepdims=True) * inv_d
    c = y - mean
    if d_true != Dp:
        if y.ndim == 2:
            col = lax.broadcasted_iota(jnp.int32, (1, y.shape[1]), 1)
        else:
            col = (lax.broadcasted_iota(jnp.int32, (1,) + y.shape[1:], 1) * 128
                   + lax.broadcasted_iota(jnp.int32, (1,) + y.shape[1:], 2))
        c = jnp.where(col < d_true, c, 0.0)
    var = jnp.sum(c * c, axis=red_axes, keepdims=True) * inv_d
    return jnp.maximum(c * lax.rsqrt(var + _EPS) * g + b, 0.0)


def _mm_kernel(x_ref, w_ref, o_ref):
    o_ref[...] = jnp.dot(x_ref[...], w_ref[...],
                         preferred_element_type=jnp.float32).astype(o_ref.dtype)


def _dense_mm(x, w, row_tile, out_dtype):
    R = x.shape[0]
    D = w.shape[1]
    return pl.pallas_call(
        _mm_kernel,
        out_shape=jax.ShapeDtypeStruct((R, D), out_dtype),
        grid=(R // row_tile,),
        in_specs=[pl.BlockSpec((row_tile, x.shape[1]), lambda i: (i, 0)),
                  pl.BlockSpec(w.shape, lambda i: (0, 0))],
        out_specs=pl.BlockSpec((row_tile, D), lambda i: (i, 0)),
        compiler_params=pltpu.CompilerParams(
            dimension_semantics=("parallel",),
            vmem_limit_bytes=_VMEM_LIMIT),
        cost_estimate=pl.CostEstimate(
            flops=2 * R * x.shape[1] * D, transcendentals=0,
            bytes_accessed=x.size * 2 + R * D * jnp.dtype(out_dtype).itemsize),
    )(x, w)


def _gather_add_kernel(src_ref, perm_ref, t_ref, u_ref, o_ref, *, te):
    base = pl.program_id(0) * te

    for j in range(te):
        s = src_ref[base + j]
        p = perm_ref[base + j]
        o_ref[j] = t_ref[s] + u_ref[p].astype(jnp.float32)


def _scatter_update_kernel(lo_ref, hi_ref, dst_ref, msg_ref, g1_ref, b1_ref,
                           h_ref, w2a_ref, w2b_ref, g_ref, b_ref, o_ref,
                           acc_ref, *, d_true, tn, te):
    ni = pl.program_id(0)
    ei = pl.program_id(1)

    @pl.when(ei == 0)
    def _():
        acc_ref[...] = jnp.zeros_like(acc_ref)

    lo = lo_ref[ni]
    hi = hi_ref[ni]

    @pl.when(jnp.logical_and(ei >= lo, ei <= hi))
    def _():
        msgb = _layernorm_relu(msg_ref[...], g1_ref[...], b1_ref[...],
                               d_true).astype(jnp.bfloat16)
        node_ids = ni * tn + lax.broadcasted_iota(jnp.int32, (tn, te), 0)
        onehot = (node_ids == dst_ref[...]).astype(jnp.bfloat16)
        acc_ref[...] += jnp.dot(onehot, msgb,
                                preferred_element_type=jnp.float32)

    @pl.when(ei == pl.num_programs(1) - 1)
    def _():
        h32 = h_ref[...]
        y = jnp.dot(h32.astype(jnp.bfloat16), w2a_ref[...],
                    preferred_element_type=jnp.float32)
        y = y + jnp.dot(acc_ref[...].astype(jnp.bfloat16), w2b_ref[...],
                        preferred_element_type=jnp.float32)
        yn = _layernorm_relu(y, g_ref[...], b_ref[...], d_true)
        o_ref[...] = yn + h32


def kernel(H, idx, X_e, W1, W2, g1, b1, g2, b2):
    H = H.astype(jnp.float32)
    X_e = X_e.astype(jnp.float32)
    N, d_h = H.shape
    E, d_e = X_e.shape
    W1 = W1.astype(jnp.float32)
    W2 = W2.astype(jnp.float32)
    hidden = W1.shape[1]
    Dp = _round_up(hidden, _LANE)
    mid = Dp // _LANE

    te = min(_EDGE_TILE, _round_up(E, _LANE))
    tn = min(_NODE_TILE, _round_up(N, 8))
    E_pad = _round_up(E, te)
    N_pad = _round_up(N, tn)
    T_e = E_pad // te
    T_n = N_pad // tn

    src = idx[0].astype(jnp.int32)
    dst = idx[1].astype(jnp.int32)

    dst_s, perm = lax.sort_key_val(dst, lax.iota(jnp.int32, E))
    src_s = jnp.take(src, perm)
    src_sp = jnp.pad(src_s, (0, E_pad - E))
    perm_p = jnp.pad(perm, (0, E_pad - E))

    d_ep = _round_up(d_e, _LANE)
    w1a = _pad2d(W1[:d_h], cols=Dp).astype(jnp.bfloat16)
    w1b = _pad2d(W1[d_h:], rows=d_ep, cols=Dp).astype(jnp.bfloat16)
    h_bf = H.astype(jnp.bfloat16)
    xe_bf = _pad2d(X_e.astype(jnp.bfloat16), rows=E_pad, cols=d_ep)

    T = _dense_mm(h_bf, w1a, min(1024, N), jnp.float32)
    U = _dense_mm(xe_bf, w1b, te, jnp.bfloat16)
    T3 = T.reshape(N, mid, _LANE)
    U3 = U.reshape(E_pad, mid, _LANE)

    ta = min(512, te)
    msg3 = pl.pallas_call(
        functools.partial(_gather_add_kernel, te=ta),
        out_shape=jax.ShapeDtypeStruct((E_pad, mid, _LANE), jnp.float32),
        grid_spec=pltpu.PrefetchScalarGridSpec(
            num_scalar_prefetch=2,
            grid=(E_pad // ta,),
            in_specs=[
                pl.BlockSpec((N, mid, _LANE), lambda i, s_r, p_r: (0, 0, 0)),
                pl.BlockSpec((E_pad, mid, _LANE), lambda i, s_r, p_r: (0, 0, 0)),
            ],
            out_specs=pl.BlockSpec((ta, mid, _LANE), lambda i, s_r, p_r: (i, 0, 0)),
        ),
        compiler_params=pltpu.CompilerParams(
            dimension_semantics=("parallel",),
            vmem_limit_bytes=_VMEM_LIMIT),
        cost_estimate=pl.CostEstimate(
            flops=2 * E_pad * Dp, transcendentals=0,
            bytes_accessed=E_pad * Dp * 8 + N * Dp * 4),
    )(src_sp, perm_p, T3, U3)
    msg = msg3.reshape(E_pad, Dp)
    g1p = _pad2d(g1.reshape(1, -1).astype(jnp.float32), cols=Dp)
    b1p = _pad2d(b1.reshape(1, -1).astype(jnp.float32), cols=Dp)

    dst_sp = jnp.pad(dst_s, (0, E_pad - E), constant_values=N_pad)
    tile_min = dst_sp[::te]
    tile_max = dst_sp[te - 1::te]
    starts = jnp.arange(T_n, dtype=jnp.int32) * tn
    lo = jnp.searchsorted(tile_max, starts, side='left').astype(jnp.int32)
    hi = (jnp.searchsorted(tile_min, starts + tn - 1, side='right')
          .astype(jnp.int32) - 1)
    empty = lo > hi
    lo_c = jnp.where(empty, 0, lo)
    hi_c = jnp.where(empty, -1, hi)

    h_pad = _pad2d(H, rows=N_pad, cols=Dp)
    w2a = _pad2d(W2[:d_h], rows=Dp, cols=Dp).astype(jnp.bfloat16)
    w2b = _pad2d(W2[d_h:], rows=Dp, cols=Dp).astype(jnp.bfloat16)
    g2p = _pad2d(g2.reshape(1, -1).astype(jnp.float32), cols=Dp)
    b2p = _pad2d(b2.reshape(1, -1).astype(jnp.float32), cols=Dp)

    def _clamp(ei, lo_r, hi_r, ni):
        return jnp.clip(ei, lo_r[ni], jnp.maximum(hi_r[ni], lo_r[ni]))

    out = pl.pallas_call(
        functools.partial(_scatter_update_kernel, d_true=hidden, tn=tn, te=te),
        out_shape=jax.ShapeDtypeStruct((N_pad, Dp), jnp.float32),
        grid_spec=pltpu.PrefetchScalarGridSpec(
            num_scalar_prefetch=2,
            grid=(T_n, T_e),
            in_specs=[
                pl.BlockSpec((1, te),
                             lambda ni, ei, lo_r, hi_r: (0, _clamp(ei, lo_r, hi_r, ni))),
                pl.BlockSpec((te, Dp),
                             lambda ni, ei, lo_r, hi_r: (_clamp(ei, lo_r, hi_r, ni), 0)),
                pl.BlockSpec((1, Dp), lambda ni, ei, lo_r, hi_r: (0, 0)),
                pl.BlockSpec((1, Dp), lambda ni, ei, lo_r, hi_r: (0, 0)),
                pl.BlockSpec((tn, Dp), lambda ni, ei, lo_r, hi_r: (ni, 0)),
                pl.BlockSpec((Dp, Dp), lambda ni, ei, lo_r, hi_r: (0, 0)),
                pl.BlockSpec((Dp, Dp), lambda ni, ei, lo_r, hi_r: (0, 0)),
                pl.BlockSpec((1, Dp), lambda ni, ei, lo_r, hi_r: (0, 0)),
                pl.BlockSpec((1, Dp), lambda ni, ei, lo_r, hi_r: (0, 0)),
            ],
            out_specs=pl.BlockSpec((tn, Dp), lambda ni, ei, lo_r, hi_r: (ni, 0)),
            scratch_shapes=[pltpu.VMEM((tn, Dp), jnp.float32)],
        ),
        compiler_params=pltpu.CompilerParams(
            dimension_semantics=("parallel", "arbitrary"),
            vmem_limit_bytes=_VMEM_LIMIT),
        cost_estimate=pl.CostEstimate(
            flops=2 * E_pad * 2 * tn * Dp + 2 * N_pad * 2 * Dp * Dp,
            transcendentals=N_pad,
            bytes_accessed=2 * E_pad * Dp * 2 + 2 * N_pad * Dp * 4),
    )(lo_c, hi_c, dst_sp.reshape(1, E_pad), msg, g1p, b1p, h_pad, w2a, w2b,
      g2p, b2p)

    return out[:N, :hidden]


# speedup vs baseline: 1.1084x; 1.1084x over previous
import functools

import jax
import jax.numpy as jnp
from jax import lax
from jax.experimental import pallas as pl
from jax.experimental.pallas import tpu as pltpu

_EPS = 1e-5
_LANE = 128
_VMEM_LIMIT = 60 * 1024 * 1024
_NODE_TILE = 1024
_EDGE_TILE = 1024
_GATHER_UNROLL = 8


def _round_up(x, m):
    return ((x + m - 1) // m) * m


def _pad2d(x, rows=None, cols=None):
    r = 0 if rows is None else rows - x.shape[0]
    c = 0 if cols is None else cols - x.shape[1]
    if r == 0 and c == 0:
        return x
    return jnp.pad(x, ((0, r), (0, c)))


def _layernorm_relu(y, g, b, d_true):
    red_axes = tuple(range(1, y.ndim))
    Dp = 1
    for a in red_axes:
        Dp *= y.shape[a]
    inv_d = 1.0 / float(d_true)
    mean = jnp.sum(y, axis=red_axes, keepdims=True) * inv_d
    c = y - mean
    if d_true != Dp:
        if y.ndim == 2:
            col = lax.broadcasted_iota(jnp.int32, (1, y.shape[1]), 1)
        else:
            col = (lax.broadcasted_iota(jnp.int32, (1,) + y.shape[1:], 1) * 128
                   + lax.broadcasted_iota(jnp.int32, (1,) + y.shape[1:], 2))
        c = jnp.where(col < d_true, c, 0.0)
    var = jnp.sum(c * c, axis=red_axes, keepdims=True) * inv_d
    return jnp.maximum(c * lax.rsqrt(var + _EPS) * g + b, 0.0)


def _mm_kernel(x_ref, w_ref, o_ref):
    o_ref[...] = jnp.dot(x_ref[...], w_ref[...],
                         preferred_element_type=jnp.float32).astype(o_ref.dtype)


def _dense_mm(x, w, row_tile, out_dtype):
    R = x.shape[0]
    D = w.shape[1]
    return pl.pallas_call(
        _mm_kernel,
        out_shape=jax.ShapeDtypeStruct((R, D), out_dtype),
        grid=(R // row_tile,),
        in_specs=[pl.BlockSpec((row_tile, x.shape[1]), lambda i: (i, 0)),
                  pl.BlockSpec(w.shape, lambda i: (0, 0))],
        out_specs=pl.BlockSpec((row_tile, D), lambda i: (i, 0)),
        compiler_params=pltpu.CompilerParams(
            dimension_semantics=("parallel",),
            vmem_limit_bytes=_VMEM_LIMIT),
        cost_estimate=pl.CostEstimate(
            flops=2 * R * x.shape[1] * D, transcendentals=0,
            bytes_accessed=x.size * 2 + R * D * jnp.dtype(out_dtype).itemsize),
    )(x, w)


def _gather_add_kernel(src_ref, perm_ref, t_ref, u_ref, o_ref, *, te):
    base = pl.program_id(0) * te

    for j in range(te):
        s = src_ref[base + j]
        p = perm_ref[base + j]
        o_ref[j] = t_ref[s] + u_ref[p].astype(jnp.float32)


def _scatter_update_kernel(lo_ref, hi_ref, dst_ref, msg_ref, g1_ref, b1_ref,
                           h_ref, w2a_ref, w2b_ref, g_ref, b_ref, o_ref,
                           acc_ref, *, d_true, tn, te):
    ni = pl.program_id(0)
    ei = pl.program_id(1)

    @pl.when(ei == 0)
    def _():
        acc_ref[...] = jnp.zeros_like(acc_ref)

    lo = lo_ref[ni]
    hi = hi_ref[ni]

    @pl.when(jnp.logical_and(ei >= lo, ei <= hi))
    def _():
        msgb = _layernorm_relu(msg_ref[...], g1_ref[...], b1_ref[...],
                               d_true).astype(jnp.bfloat16)
        node_ids = ni * tn + lax.broadcasted_iota(jnp.int32, (tn, te), 0)
        onehot = (node_ids == dst_ref[...]).astype(jnp.bfloat16)
        acc_ref[...] += jnp.dot(onehot, msgb,
                                preferred_element_type=jnp.float32)

    @pl.when(ei == pl.num_programs(1) - 1)
    def _():
        h32 = h_ref[...]
        y = jnp.dot(h32.astype(jnp.bfloat16), w2a_ref[...],
                    preferred_element_type=jnp.float32)
        y = y + jnp.dot(acc_ref[...].astype(jnp.bfloat16), w2b_ref[...],
                        preferred_element_type=jnp.float32)
        yn = _layernorm_relu(y, g_ref[...], b_ref[...], d_true)
        o_ref[...] = yn + h32


def kernel(H, idx, X_e, W1, W2, g1, b1, g2, b2):
    H = H.astype(jnp.float32)
    X_e = X_e.astype(jnp.float32)
    N, d_h = H.shape
    E, d_e = X_e.shape
    W1 = W1.astype(jnp.float32)
    W2 = W2.astype(jnp.float32)
    hidden = W1.shape[1]
    Dp = _round_up(hidden, _LANE)
    mid = Dp // _LANE

    te = min(_EDGE_TILE, _round_up(E, _LANE))
    tn = min(_NODE_TILE, _round_up(N, 8))
    E_pad = _round_up(E, te)
    N_pad = _round_up(N, tn)
    T_e = E_pad // te
    T_n = N_pad // tn

    src = idx[0].astype(jnp.int32)
    dst = idx[1].astype(jnp.int32)

    dst_s, perm = lax.sort_key_val(dst, lax.iota(jnp.int32, E))
    src_s = jnp.take(src, perm)
    src_sp = jnp.pad(src_s, (0, E_pad - E))
    perm_p = jnp.pad(perm, (0, E_pad - E))

    d_ep = _round_up(d_e, _LANE)
    w1a = _pad2d(W1[:d_h], cols=Dp).astype(jnp.bfloat16)
    w1b = _pad2d(W1[d_h:], rows=d_ep, cols=Dp).astype(jnp.bfloat16)
    h_bf = H.astype(jnp.bfloat16)
    xe_bf = _pad2d(X_e.astype(jnp.bfloat16), rows=E_pad, cols=d_ep)

    T = _dense_mm(h_bf, w1a, min(1024, N), jnp.float32)
    U = _dense_mm(xe_bf, w1b, te, jnp.bfloat16)
    T3 = T.reshape(N, mid, _LANE)
    U3 = U.reshape(E_pad, mid, _LANE)

    ta = min(512, te)
    msg3 = pl.pallas_call(
        functools.partial(_gather_add_kernel, te=ta),
        out_shape=jax.ShapeDtypeStruct((E_pad, mid, _LANE), jnp.float32),
        grid_spec=pltpu.PrefetchScalarGridSpec(
            num_scalar_prefetch=2,
            grid=(E_pad // ta,),
            in_specs=[
                pl.BlockSpec((N, mid, _LANE), lambda i, s_r, p_r: (0, 0, 0)),
                pl.BlockSpec((E_pad, mid, _LANE), lambda i, s_r, p_r: (0, 0, 0)),
            ],
            out_specs=pl.BlockSpec((ta, mid, _LANE), lambda i, s_r, p_r: (i, 0, 0)),
        ),
        compiler_params=pltpu.CompilerParams(
            dimension_semantics=("parallel",),
            vmem_limit_bytes=_VMEM_LIMIT),
        cost_estimate=pl.CostEstimate(
            flops=2 * E_pad * Dp, transcendentals=0,
            bytes_accessed=E_pad * Dp * 8 + N * Dp * 4),
    )(src_sp, perm_p, T3, U3)
    msg = msg3.reshape(E_pad, Dp)
    g1p = _pad2d(g1.reshape(1, -1).astype(jnp.float32), cols=Dp)
    b1p = _pad2d(b1.reshape(1, -1).astype(jnp.float32), cols=Dp)

    tb = 2 * te if E_pad % (2 * te) == 0 else te
    T_b = E_pad // tb
    dst_sp = jnp.pad(dst_s, (0, E_pad - E), constant_values=N_pad)
    tile_min = dst_sp[::tb]
    tile_max = dst_sp[tb - 1::tb]
    starts = jnp.arange(T_n, dtype=jnp.int32) * tn
    lo = jnp.searchsorted(tile_max, starts, side='left').astype(jnp.int32)
    hi = (jnp.searchsorted(tile_min, starts + tn - 1, side='right')
          .astype(jnp.int32) - 1)
    empty = lo > hi
    lo_c = jnp.where(empty, 0, lo)
    hi_c = jnp.where(empty, -1, hi)

    h_pad = _pad2d(H, rows=N_pad, cols=Dp)
    w2a = _pad2d(W2[:d_h], rows=Dp, cols=Dp).astype(jnp.bfloat16)
    w2b = _pad2d(W2[d_h:], rows=Dp, cols=Dp).astype(jnp.bfloat16)
    g2p = _pad2d(g2.reshape(1, -1).astype(jnp.float32), cols=Dp)
    b2p = _pad2d(b2.reshape(1, -1).astype(jnp.float32), cols=Dp)

    def _clamp(ei, lo_r, hi_r, ni):
        return jnp.clip(ei, lo_r[ni], jnp.maximum(hi_r[ni], lo_r[ni]))

    out = pl.pallas_call(
        functools.partial(_scatter_update_kernel, d_true=hidden, tn=tn, te=tb),
        out_shape=jax.ShapeDtypeStruct((N_pad, Dp), jnp.float32),
        grid_spec=pltpu.PrefetchScalarGridSpec(
            num_scalar_prefetch=2,
            grid=(T_n, T_b),
            in_specs=[
                pl.BlockSpec((1, tb),
                             lambda ni, ei, lo_r, hi_r: (0, _clamp(ei, lo_r, hi_r, ni))),
                pl.BlockSpec((tb, Dp),
                             lambda ni, ei, lo_r, hi_r: (_clamp(ei, lo_r, hi_r, ni), 0)),
                pl.BlockSpec((1, Dp), lambda ni, ei, lo_r, hi_r: (0, 0)),
                pl.BlockSpec((1, Dp), lambda ni, ei, lo_r, hi_r: (0, 0)),
                pl.BlockSpec((tn, Dp), lambda ni, ei, lo_r, hi_r: (ni, 0)),
                pl.BlockSpec((Dp, Dp), lambda ni, ei, lo_r, hi_r: (0, 0)),
                pl.BlockSpec((Dp, Dp), lambda ni, ei, lo_r, hi_r: (0, 0)),
                pl.BlockSpec((1, Dp), lambda ni, ei, lo_r, hi_r: (0, 0)),
                pl.BlockSpec((1, Dp), lambda ni, ei, lo_r, hi_r: (0, 0)),
            ],
            out_specs=pl.BlockSpec((tn, Dp), lambda ni, ei, lo_r, hi_r: (ni, 0)),
            scratch_shapes=[pltpu.VMEM((tn, Dp), jnp.float32)],
        ),
        compiler_params=pltpu.CompilerParams(
            dimension_semantics=("parallel", "arbitrary"),
            vmem_limit_bytes=_VMEM_LIMIT),
        cost_estimate=pl.CostEstimate(
            flops=2 * E_pad * 2 * tn * Dp + 2 * N_pad * 2 * Dp * Dp,
            transcendentals=N_pad,
            bytes_accessed=2 * E_pad * Dp * 2 + 2 * N_pad * Dp * 4),
    )(lo_c, hi_c, dst_sp.reshape(1, E_pad), msg, g1p, b1p, h_pad, w2a, w2b,
      g2p, b2p)

    return out[:N, :hidden]


# speedup vs baseline: 1.2023x; 1.0847x over previous
import functools

import jax
import jax.numpy as jnp
from jax import lax
from jax.experimental import pallas as pl
from jax.experimental.pallas import tpu as pltpu

_EPS = 1e-5
_LANE = 128
_VMEM_LIMIT = 60 * 1024 * 1024
_NODE_TILE = 1024
_EDGE_TILE = 1024
_GATHER_UNROLL = 8


def _round_up(x, m):
    return ((x + m - 1) // m) * m


def _pad2d(x, rows=None, cols=None):
    r = 0 if rows is None else rows - x.shape[0]
    c = 0 if cols is None else cols - x.shape[1]
    if r == 0 and c == 0:
        return x
    return jnp.pad(x, ((0, r), (0, c)))


def _layernorm_relu(y, g, b, d_true):
    red_axes = tuple(range(1, y.ndim))
    Dp = 1
    for a in red_axes:
        Dp *= y.shape[a]
    inv_d = 1.0 / float(d_true)
    mean = jnp.sum(y, axis=red_axes, keepdims=True) * inv_d
    c = y - mean
    if d_true != Dp:
        if y.ndim == 2:
            col = lax.broadcasted_iota(jnp.int32, (1, y.shape[1]), 1)
        else:
            col = (lax.broadcasted_iota(jnp.int32, (1,) + y.shape[1:], 1) * 128
                   + lax.broadcasted_iota(jnp.int32, (1,) + y.shape[1:], 2))
        c = jnp.where(col < d_true, c, 0.0)
    var = jnp.sum(c * c, axis=red_axes, keepdims=True) * inv_d
    return jnp.maximum(c * lax.rsqrt(var + _EPS) * g + b, 0.0)


def _mm_kernel(x_ref, w_ref, o_ref):
    y = jnp.dot(x_ref[...], w_ref[...], preferred_element_type=jnp.float32)
    mid = o_ref.shape[1]
    for m in range(mid):
        o_ref[:, m, :] = y[:, m * _LANE:(m + 1) * _LANE].astype(o_ref.dtype)


def _dense_mm3(x, w, row_tile, out_dtype):
    R = x.shape[0]
    D = w.shape[1]
    mid = D // _LANE
    return pl.pallas_call(
        _mm_kernel,
        out_shape=jax.ShapeDtypeStruct((R, mid, _LANE), out_dtype),
        grid=(R // row_tile,),
        in_specs=[pl.BlockSpec((row_tile, x.shape[1]), lambda i: (i, 0)),
                  pl.BlockSpec(w.shape, lambda i: (0, 0))],
        out_specs=pl.BlockSpec((row_tile, mid, _LANE), lambda i: (i, 0, 0)),
        compiler_params=pltpu.CompilerParams(
            dimension_semantics=("parallel",),
            vmem_limit_bytes=_VMEM_LIMIT),
        cost_estimate=pl.CostEstimate(
            flops=2 * R * x.shape[1] * D, transcendentals=0,
            bytes_accessed=x.size * 2 + R * D * jnp.dtype(out_dtype).itemsize),
    )(x, w)


def _gather_add_kernel(src_ref, perm_ref, t_ref, u_ref, o_ref, *, te):
    base = pl.program_id(0) * te

    for j in range(te):
        s = src_ref[base + j]
        p = perm_ref[base + j]
        o_ref[j] = t_ref[s] + u_ref[p].astype(jnp.float32)


def _scatter_update_kernel(lo_ref, hi_ref, dst_ref, msg_ref, g1_ref, b1_ref,
                           h_ref, w2a_ref, w2b_ref, g_ref, b_ref, o_ref,
                           acc_ref, *, d_true, tn, te):
    ni = pl.program_id(0)
    ei = pl.program_id(1)

    @pl.when(ei == 0)
    def _():
        acc_ref[...] = jnp.zeros_like(acc_ref)

    lo = lo_ref[ni]
    hi = hi_ref[ni]

    @pl.when(jnp.logical_and(ei >= lo, ei <= hi))
    def _():
        msgb = _layernorm_relu(msg_ref[...], g1_ref[...], b1_ref[...],
                               d_true).astype(jnp.bfloat16)
        node_ids = ni * tn + lax.broadcasted_iota(jnp.int32, (tn, te), 0)
        onehot = (node_ids == dst_ref[...]).astype(jnp.bfloat16)
        acc_ref[...] += jnp.dot(onehot, msgb,
                                preferred_element_type=jnp.float32)

    @pl.when(ei == pl.num_programs(1) - 1)
    def _():
        h32 = h_ref[...]
        y = jnp.dot(h32.astype(jnp.bfloat16), w2a_ref[...],
                    preferred_element_type=jnp.float32)
        y = y + jnp.dot(acc_ref[...].astype(jnp.bfloat16), w2b_ref[...],
                        preferred_element_type=jnp.float32)
        yn = _layernorm_relu(y, g_ref[...], b_ref[...], d_true)
        o_ref[...] = yn + h32


def kernel(H, idx, X_e, W1, W2, g1, b1, g2, b2):
    H = H.astype(jnp.float32)
    X_e = X_e.astype(jnp.float32)
    N, d_h = H.shape
    E, d_e = X_e.shape
    W1 = W1.astype(jnp.float32)
    W2 = W2.astype(jnp.float32)
    hidden = W1.shape[1]
    Dp = _round_up(hidden, _LANE)
    mid = Dp // _LANE

    te = min(_EDGE_TILE, _round_up(E, _LANE))
    tn = min(_NODE_TILE, _round_up(N, 8))
    E_pad = _round_up(E, te)
    N_pad = _round_up(N, tn)
    T_e = E_pad // te
    T_n = N_pad // tn

    src = idx[0].astype(jnp.int32)
    dst = idx[1].astype(jnp.int32)

    dst_s, perm = lax.sort_key_val(dst, lax.iota(jnp.int32, E))
    src_s = jnp.take(src, perm)
    src_sp = jnp.pad(src_s, (0, E_pad - E))
    perm_p = jnp.pad(perm, (0, E_pad - E))

    d_ep = _round_up(d_e, _LANE)
    w1a = _pad2d(W1[:d_h], cols=Dp).astype(jnp.bfloat16)
    w1b = _pad2d(W1[d_h:], rows=d_ep, cols=Dp).astype(jnp.bfloat16)
    h_bf = H.astype(jnp.bfloat16)
    xe_bf = _pad2d(X_e.astype(jnp.bfloat16), rows=E_pad, cols=d_ep)

    T3 = _dense_mm3(h_bf, w1a, min(1024, N), jnp.float32)
    U3 = _dense_mm3(xe_bf, w1b, te, jnp.bfloat16)

    ta = min(1024, te)
    msg3 = pl.pallas_call(
        functools.partial(_gather_add_kernel, te=ta),
        out_shape=jax.ShapeDtypeStruct((E_pad, mid, _LANE), jnp.float32),
        grid_spec=pltpu.PrefetchScalarGridSpec(
            num_scalar_prefetch=2,
            grid=(E_pad // ta,),
            in_specs=[
                pl.BlockSpec((N, mid, _LANE), lambda i, s_r, p_r: (0, 0, 0)),
                pl.BlockSpec((E_pad, mid, _LANE), lambda i, s_r, p_r: (0, 0, 0)),
            ],
            out_specs=pl.BlockSpec((ta, mid, _LANE), lambda i, s_r, p_r: (i, 0, 0)),
        ),
        compiler_params=pltpu.CompilerParams(
            dimension_semantics=("parallel",),
            vmem_limit_bytes=_VMEM_LIMIT),
        cost_estimate=pl.CostEstimate(
            flops=2 * E_pad * Dp, transcendentals=0,
            bytes_accessed=E_pad * Dp * 8 + N * Dp * 4),
    )(src_sp, perm_p, T3, U3)
    msg = msg3.reshape(E_pad, Dp)
    g1p = _pad2d(g1.reshape(1, -1).astype(jnp.float32), cols=Dp)
    b1p = _pad2d(b1.reshape(1, -1).astype(jnp.float32), cols=Dp)

    tb = 2 * te if E_pad % (2 * te) == 0 else te
    T_b = E_pad // tb
    dst_sp = jnp.pad(dst_s, (0, E_pad - E), constant_values=N_pad)
    tile_min = dst_sp[::tb]
    tile_max = dst_sp[tb - 1::tb]
    starts = jnp.arange(T_n, dtype=jnp.int32) * tn
    lo = jnp.searchsorted(tile_max, starts, side='left').astype(jnp.int32)
    hi = (jnp.searchsorted(tile_min, starts + tn - 1, side='right')
          .astype(jnp.int32) - 1)
    empty = lo > hi
    lo_c = jnp.where(empty, 0, lo)
    hi_c = jnp.where(empty, -1, hi)

    h_pad = _pad2d(H, rows=N_pad, cols=Dp)
    w2a = _pad2d(W2[:d_h], rows=Dp, cols=Dp).astype(jnp.bfloat16)
    w2b = _pad2d(W2[d_h:], rows=Dp, cols=Dp).astype(jnp.bfloat16)
    g2p = _pad2d(g2.reshape(1, -1).astype(jnp.float32), cols=Dp)
    b2p = _pad2d(b2.reshape(1, -1).astype(jnp.float32), cols=Dp)

    def _clamp(ei, lo_r, hi_r, ni):
        return jnp.clip(ei, lo_r[ni], jnp.maximum(hi_r[ni], lo_r[ni]))

    out = pl.pallas_call(
        functools.partial(_scatter_update_kernel, d_true=hidden, tn=tn, te=tb),
        out_shape=jax.ShapeDtypeStruct((N_pad, Dp), jnp.float32),
        grid_spec=pltpu.PrefetchScalarGridSpec(
            num_scalar_prefetch=2,
            grid=(T_n, T_b),
            in_specs=[
                pl.BlockSpec((1, tb),
                             lambda ni, ei, lo_r, hi_r: (0, _clamp(ei, lo_r, hi_r, ni))),
                pl.BlockSpec((tb, Dp),
                             lambda ni, ei, lo_r, hi_r: (_clamp(ei, lo_r, hi_r, ni), 0)),
                pl.BlockSpec((1, Dp), lambda ni, ei, lo_r, hi_r: (0, 0)),
                pl.BlockSpec((1, Dp), lambda ni, ei, lo_r, hi_r: (0, 0)),
                pl.BlockSpec((tn, Dp), lambda ni, ei, lo_r, hi_r: (ni, 0)),
                pl.BlockSpec((Dp, Dp), lambda ni, ei, lo_r, hi_r: (0, 0)),
                pl.BlockSpec((Dp, Dp), lambda ni, ei, lo_r, hi_r: (0, 0)),
                pl.BlockSpec((1, Dp), lambda ni, ei, lo_r, hi_r: (0, 0)),
                pl.BlockSpec((1, Dp), lambda ni, ei, lo_r, hi_r: (0, 0)),
            ],
            out_specs=pl.BlockSpec((tn, Dp), lambda ni, ei, lo_r, hi_r: (ni, 0)),
            scratch_shapes=[pltpu.VMEM((tn, Dp), jnp.float32)],
        ),
        compiler_params=pltpu.CompilerParams(
            dimension_semantics=("parallel", "arbitrary"),
            vmem_limit_bytes=_VMEM_LIMIT),
        cost_estimate=pl.CostEstimate(
            flops=2 * E_pad * 2 * tn * Dp + 2 * N_pad * 2 * Dp * Dp,
            transcendentals=N_pad,
            bytes_accessed=2 * E_pad * Dp * 2 + 2 * N_pad * Dp * 4),
    )(lo_c, hi_c, dst_sp.reshape(1, E_pad), msg, g1p, b1p, h_pad, w2a, w2b,
      g2p, b2p)

    return out[:N, :hidden]


# speedup vs baseline: 1.2783x; 1.0632x over previous
import functools

import jax
import jax.numpy as jnp
from jax import lax
from jax.experimental import pallas as pl
from jax.experimental.pallas import tpu as pltpu

_EPS = 1e-5
_LANE = 128
_VMEM_LIMIT = 60 * 1024 * 1024
_NODE_TILE = 1024
_EDGE_TILE = 1024
_GATHER_UNROLL = 8


def _round_up(x, m):
    return ((x + m - 1) // m) * m


def _pad2d(x, rows=None, cols=None):
    r = 0 if rows is None else rows - x.shape[0]
    c = 0 if cols is None else cols - x.shape[1]
    if r == 0 and c == 0:
        return x
    return jnp.pad(x, ((0, r), (0, c)))


def _layernorm_relu(y, g, b, d_true):
    red_axes = tuple(range(1, y.ndim))
    Dp = 1
    for a in red_axes:
        Dp *= y.shape[a]
    inv_d = 1.0 / float(d_true)
    mean = jnp.sum(y, axis=red_axes, keepdims=True) * inv_d
    c = y - mean
    if d_true != Dp:
        if y.ndim == 2:
            col = lax.broadcasted_iota(jnp.int32, (1, y.shape[1]), 1)
        else:
            col = (lax.broadcasted_iota(jnp.int32, (1,) + y.shape[1:], 1) * 128
                   + lax.broadcasted_iota(jnp.int32, (1,) + y.shape[1:], 2))
        c = jnp.where(col < d_true, c, 0.0)
    var = jnp.sum(c * c, axis=red_axes, keepdims=True) * inv_d
    return jnp.maximum(c * lax.rsqrt(var + _EPS) * g + b, 0.0)


def _mm_kernel(x_ref, w_ref, o_ref):
    y = jnp.dot(x_ref[...], w_ref[...], preferred_element_type=jnp.float32)
    mid = o_ref.shape[1]
    for m in range(mid):
        o_ref[:, m, :] = y[:, m * _LANE:(m + 1) * _LANE].astype(o_ref.dtype)


def _dense_mm3(x, w, row_tile, out_dtype):
    R = x.shape[0]
    D = w.shape[1]
    mid = D // _LANE
    return pl.pallas_call(
        _mm_kernel,
        out_shape=jax.ShapeDtypeStruct((R, mid, _LANE), out_dtype),
        grid=(R // row_tile,),
        in_specs=[pl.BlockSpec((row_tile, x.shape[1]), lambda i: (i, 0)),
                  pl.BlockSpec(w.shape, lambda i: (0, 0))],
        out_specs=pl.BlockSpec((row_tile, mid, _LANE), lambda i: (i, 0, 0)),
        compiler_params=pltpu.CompilerParams(
            dimension_semantics=("parallel",),
            vmem_limit_bytes=_VMEM_LIMIT),
        cost_estimate=pl.CostEstimate(
            flops=2 * R * x.shape[1] * D, transcendentals=0,
            bytes_accessed=x.size * 2 + R * D * jnp.dtype(out_dtype).itemsize),
    )(x, w)


def _gather_add_kernel(src_ref, perm_ref, t_ref, u_ref, o_ref, *, te):
    base = pl.program_id(0) * te

    for j in range(te):
        s = src_ref[base + j]
        p = perm_ref[base + j]
        o_ref[j] = (t_ref[s] + u_ref[p].astype(jnp.float32)).astype(o_ref.dtype)


def _scatter_update_kernel(lo_ref, hi_ref, dst_ref, msg_ref, g1_ref, b1_ref,
                           h_ref, w2a_ref, w2b_ref, g_ref, b_ref, o_ref,
                           acc_ref, *, d_true, tn, te):
    ni = pl.program_id(0)
    ei = pl.program_id(1)

    @pl.when(ei == 0)
    def _():
        acc_ref[...] = jnp.zeros_like(acc_ref)

    lo = lo_ref[ni]
    hi = hi_ref[ni]

    @pl.when(jnp.logical_and(ei >= lo, ei <= hi))
    def _():
        msgb = _layernorm_relu(msg_ref[...].astype(jnp.float32), g1_ref[...],
                               b1_ref[...], d_true).astype(jnp.bfloat16)
        node_ids = ni * tn + lax.broadcasted_iota(jnp.int32, (tn, te), 0)
        onehot = (node_ids == dst_ref[...]).astype(jnp.bfloat16)
        acc_ref[...] += jnp.dot(onehot, msgb,
                                preferred_element_type=jnp.float32)

    @pl.when(ei == pl.num_programs(1) - 1)
    def _():
        h32 = h_ref[...]
        y = jnp.dot(h32.astype(jnp.bfloat16), w2a_ref[...],
                    preferred_element_type=jnp.float32)
        y = y + jnp.dot(acc_ref[...].astype(jnp.bfloat16), w2b_ref[...],
                        preferred_element_type=jnp.float32)
        yn = _layernorm_relu(y, g_ref[...], b_ref[...], d_true)
        o_ref[...] = yn + h32


def kernel(H, idx, X_e, W1, W2, g1, b1, g2, b2):
    H = H.astype(jnp.float32)
    X_e = X_e.astype(jnp.float32)
    N, d_h = H.shape
    E, d_e = X_e.shape
    W1 = W1.astype(jnp.float32)
    W2 = W2.astype(jnp.float32)
    hidden = W1.shape[1]
    Dp = _round_up(hidden, _LANE)
    mid = Dp // _LANE

    te = min(_EDGE_TILE, _round_up(E, _LANE))
    tn = min(_NODE_TILE, _round_up(N, 8))
    E_pad = _round_up(E, te)
    N_pad = _round_up(N, tn)
    T_e = E_pad // te
    T_n = N_pad // tn

    src = idx[0].astype(jnp.int32)
    dst = idx[1].astype(jnp.int32)

    dst_s, perm = lax.sort_key_val(dst, lax.iota(jnp.int32, E))
    src_s = jnp.take(src, perm)
    src_sp = jnp.pad(src_s, (0, E_pad - E))
    perm_p = jnp.pad(perm, (0, E_pad - E))

    d_ep = _round_up(d_e, _LANE)
    w1a = _pad2d(W1[:d_h], cols=Dp).astype(jnp.bfloat16)
    w1b = _pad2d(W1[d_h:], rows=d_ep, cols=Dp).astype(jnp.bfloat16)
    h_bf = H.astype(jnp.bfloat16)
    xe_bf = _pad2d(X_e.astype(jnp.bfloat16), rows=E_pad, cols=d_ep)

    T3 = _dense_mm3(h_bf, w1a, min(1024, N), jnp.float32)
    U3 = _dense_mm3(xe_bf, w1b, te, jnp.bfloat16)

    ta = min(1024, te)
    msg3 = pl.pallas_call(
        functools.partial(_gather_add_kernel, te=ta),
        out_shape=jax.ShapeDtypeStruct((E_pad, mid, _LANE), jnp.bfloat16),
        grid_spec=pltpu.PrefetchScalarGridSpec(
            num_scalar_prefetch=2,
            grid=(E_pad // ta,),
            in_specs=[
                pl.BlockSpec((N, mid, _LANE), lambda i, s_r, p_r: (0, 0, 0)),
                pl.BlockSpec((E_pad, mid, _LANE), lambda i, s_r, p_r: (0, 0, 0)),
            ],
            out_specs=pl.BlockSpec((ta, mid, _LANE), lambda i, s_r, p_r: (i, 0, 0)),
        ),
        compiler_params=pltpu.CompilerParams(
            dimension_semantics=("parallel",),
            vmem_limit_bytes=_VMEM_LIMIT),
        cost_estimate=pl.CostEstimate(
            flops=2 * E_pad * Dp, transcendentals=0,
            bytes_accessed=E_pad * Dp * 8 + N * Dp * 4),
    )(src_sp, perm_p, T3, U3)
    msg = msg3.reshape(E_pad, Dp)
    g1p = _pad2d(g1.reshape(1, -1).astype(jnp.float32), cols=Dp)
    b1p = _pad2d(b1.reshape(1, -1).astype(jnp.float32), cols=Dp)

    tb = 2 * te if E_pad % (2 * te) == 0 else te
    T_b = E_pad // tb
    dst_sp = jnp.pad(dst_s, (0, E_pad - E), constant_values=N_pad)
    tile_min = dst_sp[::tb]
    tile_max = dst_sp[tb - 1::tb]
    starts = jnp.arange(T_n, dtype=jnp.int32) * tn
    lo = jnp.searchsorted(tile_max, starts, side='left').astype(jnp.int32)
    hi = (jnp.searchsorted(tile_min, starts + tn - 1, side='right')
          .astype(jnp.int32) - 1)
    empty = lo > hi
    lo_c = jnp.where(empty, 0, lo)
    hi_c = jnp.where(empty, -1, hi)

    h_pad = _pad2d(H, rows=N_pad, cols=Dp)
    w2a = _pad2d(W2[:d_h], rows=Dp, cols=Dp).astype(jnp.bfloat16)
    w2b = _pad2d(W2[d_h:], rows=Dp, cols=Dp).astype(jnp.bfloat16)
    g2p = _pad2d(g2.reshape(1, -1).astype(jnp.float32), cols=Dp)
    b2p = _pad2d(b2.reshape(1, -1).astype(jnp.float32), cols=Dp)

    def _clamp(ei, lo_r, hi_r, ni):
        return jnp.clip(ei, lo_r[ni], jnp.maximum(hi_r[ni], lo_r[ni]))

    out = pl.pallas_call(
        functools.partial(_scatter_update_kernel, d_true=hidden, tn=tn, te=tb),
        out_shape=jax.ShapeDtypeStruct((N_pad, Dp), jnp.float32),
        grid_spec=pltpu.PrefetchScalarGridSpec(
            num_scalar_prefetch=2,
            grid=(T_n, T_b),
            in_specs=[
                pl.BlockSpec((1, tb),
                             lambda ni, ei, lo_r, hi_r: (0, _clamp(ei, lo_r, hi_r, ni))),
                pl.BlockSpec((tb, Dp),
                             lambda ni, ei, lo_r, hi_r: (_clamp(ei, lo_r, hi_r, ni), 0)),
                pl.BlockSpec((1, Dp), lambda ni, ei, lo_r, hi_r: (0, 0)),
                pl.BlockSpec((1, Dp), lambda ni, ei, lo_r, hi_r: (0, 0)),
                pl.BlockSpec((tn, Dp), lambda ni, ei, lo_r, hi_r: (ni, 0)),
                pl.BlockSpec((Dp, Dp), lambda ni, ei, lo_r, hi_r: (0, 0)),
                pl.BlockSpec((Dp, Dp), lambda ni, ei, lo_r, hi_r: (0, 0)),
                pl.BlockSpec((1, Dp), lambda ni, ei, lo_r, hi_r: (0, 0)),
                pl.BlockSpec((1, Dp), lambda ni, ei, lo_r, hi_r: (0, 0)),
            ],
            out_specs=pl.BlockSpec((tn, Dp), lambda ni, ei, lo_r, hi_r: (ni, 0)),
            scratch_shapes=[pltpu.VMEM((tn, Dp), jnp.float32)],
        ),
        compiler_params=pltpu.CompilerParams(
            dimension_semantics=("parallel", "arbitrary"),
            vmem_limit_bytes=_VMEM_LIMIT),
        cost_estimate=pl.CostEstimate(
            flops=2 * E_pad * 2 * tn * Dp + 2 * N_pad * 2 * Dp * Dp,
            transcendentals=N_pad,
            bytes_accessed=2 * E_pad * Dp * 2 + 2 * N_pad * Dp * 4),
    )(lo_c, hi_c, dst_sp.reshape(1, E_pad), msg, g1p, b1p, h_pad, w2a, w2b,
      g2p, b2p)

    return out[:N, :hidden]


# speedup vs baseline: 1.3811x; 1.0804x over previous
import functools

import jax
import jax.numpy as jnp
from jax import lax
from jax.experimental import pallas as pl
from jax.experimental.pallas import tpu as pltpu

_EPS = 1e-5
_LANE = 128
_VMEM_LIMIT = 60 * 1024 * 1024
_NODE_TILE = 1024
_EDGE_TILE = 1024
_GATHER_UNROLL = 8


def _round_up(x, m):
    return ((x + m - 1) // m) * m


def _pad2d(x, rows=None, cols=None):
    r = 0 if rows is None else rows - x.shape[0]
    c = 0 if cols is None else cols - x.shape[1]
    if r == 0 and c == 0:
        return x
    return jnp.pad(x, ((0, r), (0, c)))


def _layernorm_relu(y, g, b, d_true):
    red_axes = tuple(range(1, y.ndim))
    Dp = 1
    for a in red_axes:
        Dp *= y.shape[a]
    inv_d = 1.0 / float(d_true)
    mean = jnp.sum(y, axis=red_axes, keepdims=True) * inv_d
    c = y - mean
    if d_true != Dp:
        if y.ndim == 2:
            col = lax.broadcasted_iota(jnp.int32, (1, y.shape[1]), 1)
        else:
            col = (lax.broadcasted_iota(jnp.int32, (1,) + y.shape[1:], 1) * 128
                   + lax.broadcasted_iota(jnp.int32, (1,) + y.shape[1:], 2))
        c = jnp.where(col < d_true, c, 0.0)
    var = jnp.sum(c * c, axis=red_axes, keepdims=True) * inv_d
    return jnp.maximum(c * lax.rsqrt(var + _EPS) * g + b, 0.0)


def _mm_kernel(x_ref, w_ref, o_ref):
    y = jnp.dot(x_ref[...].astype(jnp.bfloat16), w_ref[...],
                preferred_element_type=jnp.float32)
    mid = o_ref.shape[1]
    for m in range(mid):
        o_ref[:, m, :] = y[:, m * _LANE:(m + 1) * _LANE].astype(o_ref.dtype)


def _dense_mm3(x, w, row_tile, out_dtype):
    R = x.shape[0]
    D = w.shape[1]
    mid = D // _LANE
    return pl.pallas_call(
        _mm_kernel,
        out_shape=jax.ShapeDtypeStruct((R, mid, _LANE), out_dtype),
        grid=(R // row_tile,),
        in_specs=[pl.BlockSpec((row_tile, x.shape[1]), lambda i: (i, 0)),
                  pl.BlockSpec(w.shape, lambda i: (0, 0))],
        out_specs=pl.BlockSpec((row_tile, mid, _LANE), lambda i: (i, 0, 0)),
        compiler_params=pltpu.CompilerParams(
            dimension_semantics=("parallel",),
            vmem_limit_bytes=_VMEM_LIMIT),
        cost_estimate=pl.CostEstimate(
            flops=2 * R * x.shape[1] * D, transcendentals=0,
            bytes_accessed=x.size * 2 + R * D * jnp.dtype(out_dtype).itemsize),
    )(x, w)


def _gather_add_kernel(src_ref, perm_ref, t_ref, u_ref, o_ref, *, te):
    base = pl.program_id(0) * te

    for j in range(te):
        s = src_ref[base + j]
        p = perm_ref[base + j]
        o_ref[j] = (t_ref[s].astype(jnp.float32)
                    + u_ref[p].astype(jnp.float32)).astype(o_ref.dtype)


def _scatter_update_kernel(lo_ref, hi_ref, dst_ref, msg_ref, g1_ref, b1_ref,
                           h_ref, w2a_ref, w2b_ref, g_ref, b_ref, o_ref,
                           acc_ref, *, d_true, tn, te):
    ni = pl.program_id(0)
    ei = pl.program_id(1)

    @pl.when(ei == 0)
    def _():
        acc_ref[...] = jnp.zeros_like(acc_ref)

    lo = lo_ref[ni]
    hi = hi_ref[ni]

    @pl.when(jnp.logical_and(ei >= lo, ei <= hi))
    def _():
        msgb = _layernorm_relu(msg_ref[...].astype(jnp.float32), g1_ref[...],
                               b1_ref[...], d_true).astype(jnp.bfloat16)
        node_ids = ni * tn + lax.broadcasted_iota(jnp.int32, (tn, te), 0)
        onehot = (node_ids == dst_ref[...]).astype(jnp.bfloat16)
        acc_ref[...] += jnp.dot(onehot, msgb,
                                preferred_element_type=jnp.float32)

    @pl.when(ei == pl.num_programs(1) - 1)
    def _():
        h32 = h_ref[...]
        y = jnp.dot(h32.astype(jnp.bfloat16), w2a_ref[...],
                    preferred_element_type=jnp.float32)
        y = y + jnp.dot(acc_ref[...].astype(jnp.bfloat16), w2b_ref[...],
                        preferred_element_type=jnp.float32)
        yn = _layernorm_relu(y, g_ref[...], b_ref[...], d_true)
        o_ref[...] = yn + h32


def kernel(H, idx, X_e, W1, W2, g1, b1, g2, b2):
    H = H.astype(jnp.float32)
    X_e = X_e.astype(jnp.float32)
    N, d_h = H.shape
    E, d_e = X_e.shape
    W1 = W1.astype(jnp.float32)
    W2 = W2.astype(jnp.float32)
    hidden = W1.shape[1]
    Dp = _round_up(hidden, _LANE)
    mid = Dp // _LANE

    te = min(_EDGE_TILE, _round_up(E, _LANE))
    tn = min(_NODE_TILE, _round_up(N, 8))
    E_pad = _round_up(E, te)
    N_pad = _round_up(N, tn)
    T_e = E_pad // te
    T_n = N_pad // tn

    src = idx[0].astype(jnp.int32)
    dst = idx[1].astype(jnp.int32)

    dst_s, perm = lax.sort_key_val(dst, lax.iota(jnp.int32, E))
    src_s = jnp.take(src, perm)
    src_sp = jnp.pad(src_s, (0, E_pad - E))
    perm_p = jnp.pad(perm, (0, E_pad - E))

    d_ep = _round_up(d_e, _LANE)
    w1a = _pad2d(W1[:d_h], cols=Dp).astype(jnp.bfloat16)
    w1b = _pad2d(W1[d_h:], rows=d_ep, cols=Dp).astype(jnp.bfloat16)
    xe_p = _pad2d(X_e, rows=E_pad, cols=d_ep)

    T3 = _dense_mm3(H, w1a, min(1024, N), jnp.bfloat16)
    U3 = _dense_mm3(xe_p, w1b, te, jnp.bfloat16)

    ta = min(1024, te)
    msg3 = pl.pallas_call(
        functools.partial(_gather_add_kernel, te=ta),
        out_shape=jax.ShapeDtypeStruct((E_pad, mid, _LANE), jnp.bfloat16),
        grid_spec=pltpu.PrefetchScalarGridSpec(
            num_scalar_prefetch=2,
            grid=(E_pad // ta,),
            in_specs=[
                pl.BlockSpec((N, mid, _LANE), lambda i, s_r, p_r: (0, 0, 0)),
                pl.BlockSpec((E_pad, mid, _LANE), lambda i, s_r, p_r: (0, 0, 0)),
            ],
            out_specs=pl.BlockSpec((ta, mid, _LANE), lambda i, s_r, p_r: (i, 0, 0)),
        ),
        compiler_params=pltpu.CompilerParams(
            dimension_semantics=("parallel",),
            vmem_limit_bytes=_VMEM_LIMIT),
        cost_estimate=pl.CostEstimate(
            flops=2 * E_pad * Dp, transcendentals=0,
            bytes_accessed=E_pad * Dp * 8 + N * Dp * 4),
    )(src_sp, perm_p, T3, U3)
    msg = msg3.reshape(E_pad, Dp)
    g1p = _pad2d(g1.reshape(1, -1).astype(jnp.float32), cols=Dp)
    b1p = _pad2d(b1.reshape(1, -1).astype(jnp.float32), cols=Dp)

    tb = 2 * te if E_pad % (2 * te) == 0 else te
    T_b = E_pad // tb
    dst_sp = jnp.pad(dst_s, (0, E_pad - E), constant_values=N_pad)
    tile_min = dst_sp[::tb]
    tile_max = dst_sp[tb - 1::tb]
    starts = jnp.arange(T_n, dtype=jnp.int32) * tn
    lo = jnp.searchsorted(tile_max, starts, side='left').astype(jnp.int32)
    hi = (jnp.searchsorted(tile_min, starts + tn - 1, side='right')
          .astype(jnp.int32) - 1)
    empty = lo > hi
    lo_c = jnp.where(empty, 0, lo)
    hi_c = jnp.where(empty, -1, hi)

    h_pad = _pad2d(H, rows=N_pad, cols=Dp)
    w2a = _pad2d(W2[:d_h], rows=Dp, cols=Dp).astype(jnp.bfloat16)
    w2b = _pad2d(W2[d_h:], rows=Dp, cols=Dp).astype(jnp.bfloat16)
    g2p = _pad2d(g2.reshape(1, -1).astype(jnp.float32), cols=Dp)
    b2p = _pad2d(b2.reshape(1, -1).astype(jnp.float32), cols=Dp)

    def _clamp(ei, lo_r, hi_r, ni):
        return jnp.clip(ei, lo_r[ni], jnp.maximum(hi_r[ni], lo_r[ni]))

    out = pl.pallas_call(
        functools.partial(_scatter_update_kernel, d_true=hidden, tn=tn, te=tb),
        out_shape=jax.ShapeDtypeStruct((N_pad, Dp), jnp.float32),
        grid_spec=pltpu.PrefetchScalarGridSpec(
            num_scalar_prefetch=2,
            grid=(T_n, T_b),
            in_specs=[
                pl.BlockSpec((1, tb),
                             lambda ni, ei, lo_r, hi_r: (0, _clamp(ei, lo_r, hi_r, ni))),
                pl.BlockSpec((tb, Dp),
                             lambda ni, ei, lo_r, hi_r: (_clamp(ei, lo_r, hi_r, ni), 0)),
                pl.BlockSpec((1, Dp), lambda ni, ei, lo_r, hi_r: (0, 0)),
                pl.BlockSpec((1, Dp), lambda ni, ei, lo_r, hi_r: (0, 0)),
                pl.BlockSpec((tn, Dp), lambda ni, ei, lo_r, hi_r: (ni, 0)),
                pl.BlockSpec((Dp, Dp), lambda ni, ei, lo_r, hi_r: (0, 0)),
                pl.BlockSpec((Dp, Dp), lambda ni, ei, lo_r, hi_r: (0, 0)),
                pl.BlockSpec((1, Dp), lambda ni, ei, lo_r, hi_r: (0, 0)),
                pl.BlockSpec((1, Dp), lambda ni, ei, lo_r, hi_r: (0, 0)),
            ],
            out_specs=pl.BlockSpec((tn, Dp), lambda ni, ei, lo_r, hi_r: (ni, 0)),
            scratch_shapes=[pltpu.VMEM((tn, Dp), jnp.float32)],
        ),
        compiler_params=pltpu.CompilerParams(
            dimension_semantics=("parallel", "arbitrary"),
            vmem_limit_bytes=_VMEM_LIMIT),
        cost_estimate=pl.CostEstimate(
            flops=2 * E_pad * 2 * tn * Dp + 2 * N_pad * 2 * Dp * Dp,
            transcendentals=N_pad,
            bytes_accessed=2 * E_pad * Dp * 2 + 2 * N_pad * Dp * 4),
    )(lo_c, hi_c, dst_sp.reshape(1, E_pad), msg, g1p, b1p, h_pad, w2a, w2b,
      g2p, b2p)

    return out[:N, :hidden]
